# baseline (speedup 1.0000x reference)
import functools

import jax
import jax.numpy as jnp
from jax import lax
from jax.experimental import pallas as pl
from jax.experimental.pallas import tpu as pltpu
from jax.experimental.pallas import tpu_sc as plsc

_N, _D, _MAXDEG, _NCLASS, _B = 50000, 256, 32, 64, 1024
_S0, _S1 = 25, 10
_DP = _D // 2

_L = 16
_NC, _NS = 2, 16
_NW = _NC * _NS
_RPW = _B // _NW
_L1PW = _RPW * _S0
_GR = 8
_GL1 = _GR * _S0
_NGRP = _RPW // _GR
_CH2 = 80
_NB = 8
_FB = _NB * _S1


def _pipe2(n_blocks, fire, consume, bufA, semA, bufB, semB, wait):
    fire(0, bufA, semA)
    npairs = (n_blocks - 2) // 2

    def pair(i, c):
        fire(2 * i + 1, bufB, semB)
        wait(bufA, semA)
        consume(2 * i, bufA)
        fire(2 * i + 2, bufA, semA)
        wait(bufB, semB)
        consume(2 * i + 1, bufB)
        return c
    lax.fori_loop(0, npairs, pair, 0)
    k = 2 * npairs
    if n_blocks % 2 == 0:
        fire(n_blocks - 1, bufB, semB)
        wait(bufA, semA)
        consume(k, bufA)
        wait(bufB, semB)
        consume(n_blocks - 1, bufB)
    else:
        fire(n_blocks - 2, bufB, semB)
        wait(bufA, semA)
        consume(k, bufA)
        fire(n_blocks - 1, bufA, semA)
        wait(bufB, semB)
        consume(n_blocks - 2, bufB)
        wait(bufA, semA)
        consume(n_blocks - 1, bufA)


def _sc_idx_body(nbr_hbm, tn_hbm, n1f_hbm, n2idx_hbm,
                 tn_v, n1rows_v, n1f_v, n2rA, n2rB, n2idx_v,
                 semA, semB):
    wid = lax.axis_index("s") * _NC + lax.axis_index("c")
    rbase = wid * _RPW

    pltpu.sync_copy(tn_hbm.at[pl.ds(rbase, _RPW)], tn_v)
    pltpu.async_copy(nbr_hbm.at[tn_v], n1rows_v, semA).wait()

    iota = lax.broadcasted_iota(jnp.int32, (_L,), 0)

    def pack25(i, c):
        k = i * _L + iota
        vals = plsc.load_gather(n1rows_v, [k // _S0, k % _S0])
        n1f_v[pl.ds(i * _L, _L)] = vals
        return c
    lax.fori_loop(0, _L1PW // _L, pack25, 0)
    pltpu.sync_copy(n1f_v, n1f_hbm.at[pl.ds(wid * _L1PW, _L1PW)])

    def n2_fire(m, buf, sem):
        pltpu.async_copy(nbr_hbm.at[n1f_v.at[pl.ds(m * _CH2, _CH2)]],
                         buf, sem)

    def n2_wait(buf, sem):
        pltpu.make_async_copy(nbr_hbm.at[pl.ds(0, _CH2)], buf, sem).wait()

    def n2_consume(m, buf):
        def pack10(i, c):
            k = i * _L + iota
            vals = plsc.load_gather(buf, [k // _S1, k % _S1])
            n2idx_v[pl.ds(m * _CH2 * _S1 + i * _L, _L)] = vals
            return c
        lax.fori_loop(0, _CH2 * _S1 // _L, pack10, 0)

    _pipe2(_L1PW // _CH2, n2_fire, n2_consume, n2rA, semA, n2rB, semB,
           n2_wait)
    pltpu.sync_copy(n2idx_v,
                    n2idx_hbm.at[pl.ds(wid * _L1PW * _S1, _L1PW * _S1)])


_sc_idx = functools.partial(
    pl.kernel,
    out_type=(
        jax.ShapeDtypeStruct((_B * _S0,), jnp.int32),
        jax.ShapeDtypeStruct((_B * _S0 * _S1,), jnp.int32),
    ),
    mesh=plsc.VectorSubcoreMesh(core_axis_name="c", subcore_axis_name="s",
                                num_cores=_NC, num_subcores=_NS),
    compiler_params=pltpu.CompilerParams(needs_layout_passes=False,
                                         use_tc_tiling_on_sc=False),
    scratch_types=[
        pltpu.VMEM((_RPW,), jnp.int32),
        pltpu.VMEM((_RPW, _MAXDEG), jnp.int32),
        pltpu.VMEM((_L1PW,), jnp.int32),
        pltpu.VMEM((_CH2, _MAXDEG), jnp.int32),
        pltpu.VMEM((_CH2, _MAXDEG), jnp.int32),
        pltpu.VMEM((_L1PW * _S1,), jnp.int32),
        pltpu.SemaphoreType.DMA,
        pltpu.SemaphoreType.DMA,
    ],
)(_sc_idx_body)


def _acc_rows(src_ref, row0, nrows, dst_ref, dst_row):
    mask = jnp.full((_L,), -65536, dtype=jnp.int32)
    sh16 = jnp.full((_L,), 16, dtype=jnp.int32)
    for ch in range(_DP // _L):
        sl = pl.ds(ch * _L, _L)
        w = plsc.bitcast(src_ref[row0, sl], jnp.int32)
        acc_lo = plsc.bitcast(w << sh16, jnp.float32)
        acc_hi = plsc.bitcast(w & mask, jnp.float32)
        for c in range(1, nrows):
            w = plsc.bitcast(src_ref[row0 + c, sl], jnp.int32)
            acc_lo = acc_lo + plsc.bitcast(w << sh16, jnp.float32)
            acc_hi = acc_hi + plsc.bitcast(w & mask, jnp.float32)
        pk = plsc.pack(acc_lo, acc_hi, format=plsc.PackFormat.INTERLEAVED)
        dst_ref[dst_row, sl] = plsc.bitcast(pk, jnp.float32)


def _sc_feat_body(feat_hbm, tn_hbm, n1f_hbm, n2idx_hbm,
                  self1_hbm, neigh1_hbm, self0_hbm, neigh0_hbm,
                  tn_v, n1f_v, n2idx_v, big_v, tmpA, tmpB, neigh0_v,
                  semA, semB, sem0):
    wid = lax.axis_index("s") * _NC + lax.axis_index("c")
    rbase = wid * _RPW
    _ns = jax.named_scope

    pltpu.sync_copy(tn_hbm.at[pl.ds(rbase, _RPW)], tn_v)
    pltpu.async_copy(feat_hbm.at[tn_v], big_v.at[pl.ds(0, _RPW)], sem0)
    pltpu.sync_copy(n1f_hbm.at[pl.ds(wid * _L1PW, _L1PW)], n1f_v)
    pltpu.sync_copy(n2idx_hbm.at[pl.ds(wid * _L1PW * _S1, _L1PW * _S1)],
                    n2idx_v)

    pltpu.make_async_copy(feat_hbm.at[pl.ds(0, _RPW)],
                          big_v.at[pl.ds(0, _RPW)], sem0).wait()
    pltpu.sync_copy(big_v.at[pl.ds(0, _RPW)], self0_hbm.at[pl.ds(rbase, _RPW)])

    def do_group(g, c):
        lbase = g * _GL1
        growbase = (rbase + g * _GR) * _S0

        pltpu.async_copy(feat_hbm.at[n1f_v.at[pl.ds(lbase, 120)]],
                         big_v.at[pl.ds(0, 120)], semA)
        cp2 = pltpu.async_copy(feat_hbm.at[n1f_v.at[pl.ds(lbase + 120, 80)]],
                               big_v.at[pl.ds(120, 80)], semB)
        with _ns("self1wait"):
            pltpu.make_async_copy(feat_hbm.at[pl.ds(0, 120)],
                                  big_v.at[pl.ds(0, 120)], semA).wait()
            cp2.wait()
            pltpu.sync_copy(big_v, self1_hbm.at[pl.ds(growbase, _GL1)])

        def n0root(r, cc):
            _acc_rows(big_v, r * _S0, _S1, neigh0_v, g * _GR + r)
            return cc
        with _ns("n0acc"):
            lax.fori_loop(0, _GR, n0root, 0)

        def n1_fire(t, buf, sem):
            pltpu.async_copy(
                feat_hbm.at[n2idx_v.at[pl.ds((lbase + t * _NB) * _S1, _FB)]],
                buf, sem)

        def n1_wait(buf, sem):
            pltpu.make_async_copy(feat_hbm.at[pl.ds(0, _FB)], buf, sem).wait()

        def n1_consume(t, buf):
            def node(nn, cc):
                _acc_rows(buf, nn * _S1, _S1, big_v, t * _NB + nn)
                return cc
            lax.fori_loop(0, _NB, node, 0)

        with _ns("n1pipe"):
            _pipe2(_GL1 // _NB, n1_fire, n1_consume, tmpA, semA, tmpB, semB,
                   n1_wait)
        with _ns("n1flush"):
            pltpu.sync_copy(big_v, neigh1_hbm.at[pl.ds(growbase, _GL1)])
        return c
    lax.fori_loop(0, _NGRP, do_group, 0)

    pltpu.sync_copy(neigh0_v, neigh0_hbm.at[pl.ds(rbase, _RPW)])


_sc_feat = functools.partial(
    pl.kernel,
    out_type=(
        jax.ShapeDtypeStruct((_B * _S0, _DP), jnp.float32),
        jax.ShapeDtypeStruct((_B * _S0, _DP), jnp.float32),
        jax.ShapeDtypeStruct((_B, _DP), jnp.float32),
        jax.ShapeDtypeStruct((_B, _DP), jnp.float32),
    ),
    mesh=plsc.VectorSubcoreMesh(core_axis_name="c", subcore_axis_name="s",
                                num_cores=_NC, num_subcores=_NS),
    compiler_params=pltpu.CompilerParams(needs_layout_passes=False),
    scratch_types=[
        pltpu.VMEM((_RPW,), jnp.int32),
        pltpu.VMEM((_L1PW,), jnp.int32),
        pltpu.VMEM((_L1PW * _S1,), jnp.int32),
        pltpu.VMEM((_GL1, _DP), jnp.float32),
        pltpu.VMEM((_FB, _DP), jnp.float32),
        pltpu.VMEM((_FB, _DP), jnp.float32),
        pltpu.VMEM((_RPW, _DP), jnp.float32),
        pltpu.SemaphoreType.DMA,
        pltpu.SemaphoreType.DMA,
        pltpu.SemaphoreType.DMA,
    ],
)(_sc_feat_body)


_PKROWS = 2000


def _tc_pack_body(f_ref, out_ref):
    bits = lax.bitcast_convert_type(f_ref[...], jnp.int32)
    rnd = bits + jnp.int32(0x7FFF) + ((bits >> 16) & jnp.int32(1))
    lo = (rnd[:, :_DP] >> 16) & jnp.int32(0xFFFF)
    hi = rnd[:, _DP:] & jnp.int32(-65536)
    out_ref[...] = lax.bitcast_convert_type(lo | hi, jnp.float32)


def _tc_pack(feature):
    return pl.pallas_call(
        _tc_pack_body,
        grid=(_N // _PKROWS,),
        in_specs=[pl.BlockSpec((_PKROWS, _D), lambda i: (i, 0))],
        out_specs=pl.BlockSpec((_PKROWS, _DP), lambda i: (i, 0)),
        out_shape=jax.ShapeDtypeStruct((_N, _DP), jnp.float32),
    )(feature)


_R = 128


def _tc_split(x):
    b = lax.bitcast_convert_type(x, jnp.int32)
    lo = lax.bitcast_convert_type(b << 16, jnp.float32)
    hi = lax.bitcast_convert_type(b & jnp.int32(-65536), jnp.float32)
    return lo.astype(jnp.bfloat16), hi.astype(jnp.bfloat16)


def _tc_body(s1_ref, n1_ref, s0_ref, n0_ref,
             w1al_ref, w1ah_ref, w1bl_ref, w1bh_ref, w2a_ref, w2b_ref,
             out_ref):
    f32 = jnp.float32
    s1lo, s1hi = _tc_split(s1_ref[...])
    n1lo, n1hi = _tc_split(n1_ref[...])
    h = jnp.dot(s1lo, w1al_ref[...], preferred_element_type=f32)
    h = h + jnp.dot(s1hi, w1ah_ref[...], preferred_element_type=f32)
    h = h + jnp.dot(n1lo, w1bl_ref[...], preferred_element_type=f32)
    h = h + jnp.dot(n1hi, w1bh_ref[...], preferred_element_type=f32)
    h = jnp.maximum(h, 0.0)
    neigh2 = jnp.sum(h.reshape(_R, _S0, _D), axis=1)
    s0lo, s0hi = _tc_split(s0_ref[...])
    n0lo, n0hi = _tc_split(n0_ref[...])
    hs = jnp.dot(s0lo, w1al_ref[...], preferred_element_type=f32)
    hs = hs + jnp.dot(s0hi, w1ah_ref[...], preferred_element_type=f32)
    hs = hs + jnp.dot(n0lo, w1bl_ref[...], preferred_element_type=f32)
    hs = hs + jnp.dot(n0hi, w1bh_ref[...], preferred_element_type=f32)
    hs = jnp.maximum(hs, 0.0)
    logits = jnp.dot(hs, w2a_ref[...], preferred_element_type=f32)
    logits = logits + jnp.dot(neigh2, w2b_ref[...],
                              preferred_element_type=f32)
    m = jnp.max(logits, axis=-1, keepdims=True)
    e = jnp.exp(logits - m)
    out_ref[...] = e / jnp.sum(e, axis=-1, keepdims=True)


def _tc_dense(self1, neigh1, self0, neigh0,
              w1al, w1ah, w1bl, w1bh, w2a, w2b):
    grid = (_B // _R,)
    return pl.pallas_call(
        _tc_body,
        grid=grid,
        in_specs=[
            pl.BlockSpec((_R * _S0, _DP), lambda i: (i, 0)),
            pl.BlockSpec((_R * _S0, _DP), lambda i: (i, 0)),
            pl.BlockSpec((_R, _DP), lambda i: (i, 0)),
            pl.BlockSpec((_R, _DP), lambda i: (i, 0)),
            pl.BlockSpec((_DP, _D), lambda i: (0, 0)),
            pl.BlockSpec((_DP, _D), lambda i: (0, 0)),
            pl.BlockSpec((_DP, _D), lambda i: (0, 0)),
            pl.BlockSpec((_DP, _D), lambda i: (0, 0)),
            pl.BlockSpec((_D, _NCLASS), lambda i: (0, 0)),
            pl.BlockSpec((_D, _NCLASS), lambda i: (0, 0)),
        ],
        out_specs=pl.BlockSpec((_R, _NCLASS), lambda i: (i, 0)),
        out_shape=jax.ShapeDtypeStruct((_B, _NCLASS), jnp.float32),
    )(self1, neigh1, self0, neigh0, w1al, w1ah, w1bl, w1bh, w2a, w2b)


def kernel(feature, neighbor_array, train_node, W1, W2):
    fpk = _tc_pack(feature)
    n1f, n2idx = _sc_idx(neighbor_array, train_node)
    self1, neigh1, self0, neigh0 = _sc_feat(fpk, train_node, n1f, n2idx)

    w1b = W1[_D:] * (1.0 / _S1)
    w2b = W2[_D:] * (1.0 / _S0)
    bf = jnp.bfloat16
    return _tc_dense(self1, neigh1, self0, neigh0,
                     W1[:_DP].astype(bf), W1[_DP:_D].astype(bf),
                     w1b[:_DP].astype(bf), w1b[_DP:].astype(bf),
                     W2[:_D], w2b)


# speedup vs baseline: 1.0574x; 1.0574x over previous
import functools

import jax
import jax.numpy as jnp
from jax import lax
from jax.experimental import pallas as pl
from jax.experimental.pallas import tpu as pltpu
from jax.experimental.pallas import tpu_sc as plsc

_N, _D, _MAXDEG, _NCLASS, _B = 50000, 256, 32, 64, 1024
_S0, _S1 = 25, 10
_DP = _D // 2

_L = 16
_NC, _NS = 2, 16
_NW = _NC * _NS
_RPW = _B // _NW
_L1PW = _RPW * _S0
_GR = 8
_GL1 = _GR * _S0
_NGRP = _RPW // _GR
_CH2 = 80
_NB = 8
_FB = _NB * _S1


def _pipe2(n_blocks, fire, consume, bufA, semA, bufB, semB, wait):
    fire(0, bufA, semA)
    npairs = (n_blocks - 2) // 2

    def pair(i, c):
        fire(2 * i + 1, bufB, semB)
        wait(bufA, semA)
        consume(2 * i, bufA)
        fire(2 * i + 2, bufA, semA)
        wait(bufB, semB)
        consume(2 * i + 1, bufB)
        return c
    lax.fori_loop(0, npairs, pair, 0)
    k = 2 * npairs
    if n_blocks % 2 == 0:
        fire(n_blocks - 1, bufB, semB)
        wait(bufA, semA)
        consume(k, bufA)
        wait(bufB, semB)
        consume(n_blocks - 1, bufB)
    else:
        fire(n_blocks - 2, bufB, semB)
        wait(bufA, semA)
        consume(k, bufA)
        fire(n_blocks - 1, bufA, semA)
        wait(bufB, semB)
        consume(n_blocks - 2, bufB)
        wait(bufA, semA)
        consume(n_blocks - 1, bufA)


_KC = 40


def _row_fires(nbr_hbm, idx_v, idx_base, buf, sem, n):
    def fire(j, c):
        v = idx_v[pl.ds(idx_base + j, _L)][0]
        pltpu.async_copy(nbr_hbm.at[pl.ds(v, 1)], buf.at[pl.ds(j, 1)], sem)
        return c
    lax.fori_loop(0, n, fire, 0)


def _row_drain(nbr_hbm, buf, sem, n):
    def drain(j, c):
        pltpu.make_async_copy(nbr_hbm.at[pl.ds(0, 1)],
                              buf.at[pl.ds(0, 1)], sem).wait()
        return c
    lax.fori_loop(0, n, drain, 0)


def _sc_idx_body(nbr_hbm, tn_hbm, n1f_hbm, n2idx_hbm,
                 tn_v, n1rows_v, n1f_v, n2rA, n2rB, n2idx_v,
                 semA, semB):
    wid = lax.axis_index("s") * _NC + lax.axis_index("c")
    rbase = wid * _RPW

    pltpu.sync_copy(tn_hbm.at[pl.ds(rbase, _RPW)], tn_v.at[pl.ds(0, _RPW)])
    _row_fires(nbr_hbm, tn_v, 0, n1rows_v, semA, _RPW)
    _row_drain(nbr_hbm, n1rows_v, semA, _RPW)

    iota = lax.broadcasted_iota(jnp.int32, (_L,), 0)

    def pack25(i, c):
        k = i * _L + iota
        vals = plsc.load_gather(n1rows_v, [k // _S0, k % _S0])
        n1f_v[pl.ds(i * _L, _L)] = vals
        return c
    lax.fori_loop(0, _L1PW // _L, pack25, 0)
    pltpu.sync_copy(n1f_v.at[pl.ds(0, _L1PW)],
                    n1f_hbm.at[pl.ds(wid * _L1PW, _L1PW)])

    def n2_fire(m, buf, sem):
        _row_fires(nbr_hbm, n1f_v, m * _KC, buf, sem, _KC)

    def n2_wait(buf, sem):
        _row_drain(nbr_hbm, buf, sem, _KC)

    def n2_consume(m, buf):
        def pack10(i, c):
            k = i * _L + iota
            vals = plsc.load_gather(buf, [k // _S1, k % _S1])
            n2idx_v[pl.ds(m * _KC * _S1 + i * _L, _L)] = vals
            return c
        lax.fori_loop(0, _KC * _S1 // _L, pack10, 0)

    _pipe2(_L1PW // _KC, n2_fire, n2_consume, n2rA, semA, n2rB, semB,
           n2_wait)
    pltpu.sync_copy(n2idx_v,
                    n2idx_hbm.at[pl.ds(wid * _L1PW * _S1, _L1PW * _S1)])


_sc_idx = functools.partial(
    pl.kernel,
    out_type=(
        jax.ShapeDtypeStruct((_B * _S0,), jnp.int32),
        jax.ShapeDtypeStruct((_B * _S0 * _S1,), jnp.int32),
    ),
    mesh=plsc.VectorSubcoreMesh(core_axis_name="c", subcore_axis_name="s",
                                num_cores=_NC, num_subcores=_NS),
    compiler_params=pltpu.CompilerParams(needs_layout_passes=False),
    scratch_types=[
        pltpu.VMEM((_RPW + _L,), jnp.int32),
        pltpu.VMEM((_RPW, _MAXDEG), jnp.int32),
        pltpu.VMEM((_L1PW + _L,), jnp.int32),
        pltpu.VMEM((_KC, _MAXDEG), jnp.int32),
        pltpu.VMEM((_KC, _MAXDEG), jnp.int32),
        pltpu.VMEM((_L1PW * _S1,), jnp.int32),
        pltpu.SemaphoreType.DMA,
        pltpu.SemaphoreType.DMA,
    ],
)(_sc_idx_body)


def _acc_rows(src_ref, row0, nrows, dst_ref, dst_row):
    mask = jnp.full((_L,), -65536, dtype=jnp.int32)
    sh16 = jnp.full((_L,), 16, dtype=jnp.int32)
    for ch in range(_DP // _L):
        sl = pl.ds(ch * _L, _L)
        w = plsc.bitcast(src_ref[row0, sl], jnp.int32)
        acc_lo = plsc.bitcast(w << sh16, jnp.float32)
        acc_hi = plsc.bitcast(w & mask, jnp.float32)
        for c in range(1, nrows):
            w = plsc.bitcast(src_ref[row0 + c, sl], jnp.int32)
            acc_lo = acc_lo + plsc.bitcast(w << sh16, jnp.float32)
            acc_hi = acc_hi + plsc.bitcast(w & mask, jnp.float32)
        pk = plsc.pack(acc_lo, acc_hi, format=plsc.PackFormat.INTERLEAVED)
        dst_ref[dst_row, sl] = plsc.bitcast(pk, jnp.float32)


def _sc_feat_body(feat_hbm, tn_hbm, n1f_hbm, n2idx_hbm,
                  self1_hbm, neigh1_hbm, self0_hbm, neigh0_hbm,
                  tn_v, n1f_v, n2idx_v, big_v, tmpA, tmpB, neigh0_v,
                  semA, semB, sem0):
    wid = lax.axis_index("s") * _NC + lax.axis_index("c")
    rbase = wid * _RPW
    _ns = jax.named_scope

    pltpu.sync_copy(tn_hbm.at[pl.ds(rbase, _RPW)], tn_v)
    pltpu.async_copy(feat_hbm.at[tn_v], big_v.at[pl.ds(0, _RPW)], sem0)
    pltpu.sync_copy(n1f_hbm.at[pl.ds(wid * _L1PW, _L1PW)], n1f_v)
    pltpu.sync_copy(n2idx_hbm.at[pl.ds(wid * _L1PW * _S1, _L1PW * _S1)],
                    n2idx_v)

    pltpu.make_async_copy(feat_hbm.at[pl.ds(0, _RPW)],
                          big_v.at[pl.ds(0, _RPW)], sem0).wait()
    pltpu.sync_copy(big_v.at[pl.ds(0, _RPW)], self0_hbm.at[pl.ds(rbase, _RPW)])

    def do_group(g, c):
        lbase = g * _GL1
        growbase = (rbase + g * _GR) * _S0

        pltpu.async_copy(feat_hbm.at[n1f_v.at[pl.ds(lbase, 120)]],
                         big_v.at[pl.ds(0, 120)], semA)
        cp2 = pltpu.async_copy(feat_hbm.at[n1f_v.at[pl.ds(lbase + 120, 80)]],
                               big_v.at[pl.ds(120, 80)], semB)
        with _ns("self1wait"):
            pltpu.make_async_copy(feat_hbm.at[pl.ds(0, 120)],
                                  big_v.at[pl.ds(0, 120)], semA).wait()
            cp2.wait()
            pltpu.sync_copy(big_v, self1_hbm.at[pl.ds(growbase, _GL1)])

        def n0root(r, cc):
            _acc_rows(big_v, r * _S0, _S1, neigh0_v, g * _GR + r)
            return cc
        with _ns("n0acc"):
            lax.fori_loop(0, _GR, n0root, 0)

        def n1_fire(t, buf, sem):
            pltpu.async_copy(
                feat_hbm.at[n2idx_v.at[pl.ds((lbase + t * _NB) * _S1, _FB)]],
                buf, sem)

        def n1_wait(buf, sem):
            pltpu.make_async_copy(feat_hbm.at[pl.ds(0, _FB)], buf, sem).wait()

        def n1_consume(t, buf):
            def node(nn, cc):
                _acc_rows(buf, nn * _S1, _S1, big_v, t * _NB + nn)
                return cc
            lax.fori_loop(0, _NB, node, 0)

        with _ns("n1pipe"):
            _pipe2(_GL1 // _NB, n1_fire, n1_consume, tmpA, semA, tmpB, semB,
                   n1_wait)
        with _ns("n1flush"):
            pltpu.sync_copy(big_v, neigh1_hbm.at[pl.ds(growbase, _GL1)])
        return c
    lax.fori_loop(0, _NGRP, do_group, 0)

    pltpu.sync_copy(neigh0_v, neigh0_hbm.at[pl.ds(rbase, _RPW)])


_sc_feat = functools.partial(
    pl.kernel,
    out_type=(
        jax.ShapeDtypeStruct((_B * _S0, _DP), jnp.float32),
        jax.ShapeDtypeStruct((_B * _S0, _DP), jnp.float32),
        jax.ShapeDtypeStruct((_B, _DP), jnp.float32),
        jax.ShapeDtypeStruct((_B, _DP), jnp.float32),
    ),
    mesh=plsc.VectorSubcoreMesh(core_axis_name="c", subcore_axis_name="s",
                                num_cores=_NC, num_subcores=_NS),
    compiler_params=pltpu.CompilerParams(needs_layout_passes=False),
    scratch_types=[
        pltpu.VMEM((_RPW,), jnp.int32),
        pltpu.VMEM((_L1PW,), jnp.int32),
        pltpu.VMEM((_L1PW * _S1,), jnp.int32),
        pltpu.VMEM((_GL1, _DP), jnp.float32),
        pltpu.VMEM((_FB, _DP), jnp.float32),
        pltpu.VMEM((_FB, _DP), jnp.float32),
        pltpu.VMEM((_RPW, _DP), jnp.float32),
        pltpu.SemaphoreType.DMA,
        pltpu.SemaphoreType.DMA,
        pltpu.SemaphoreType.DMA,
    ],
)(_sc_feat_body)


_PKROWS = 2000


def _tc_pack_body(f_ref, out_ref):
    bits = lax.bitcast_convert_type(f_ref[...], jnp.int32)
    rnd = bits + jnp.int32(0x7FFF) + ((bits >> 16) & jnp.int32(1))
    lo = (rnd[:, :_DP] >> 16) & jnp.int32(0xFFFF)
    hi = rnd[:, _DP:] & jnp.int32(-65536)
    out_ref[...] = lax.bitcast_convert_type(lo | hi, jnp.float32)


def _tc_pack(feature):
    return pl.pallas_call(
        _tc_pack_body,
        grid=(_N // _PKROWS,),
        in_specs=[pl.BlockSpec((_PKROWS, _D), lambda i: (i, 0))],
        out_specs=pl.BlockSpec((_PKROWS, _DP), lambda i: (i, 0)),
        out_shape=jax.ShapeDtypeStruct((_N, _DP), jnp.float32),
    )(feature)


_R = 128


def _tc_split(x):
    b = lax.bitcast_convert_type(x, jnp.int32)
    lo = lax.bitcast_convert_type(b << 16, jnp.float32)
    hi = lax.bitcast_convert_type(b & jnp.int32(-65536), jnp.float32)
    return lo.astype(jnp.bfloat16), hi.astype(jnp.bfloat16)


def _tc_body(s1_ref, n1_ref, s0_ref, n0_ref,
             w1al_ref, w1ah_ref, w1bl_ref, w1bh_ref, w2a_ref, w2b_ref,
             out_ref):
    f32 = jnp.float32
    s1lo, s1hi = _tc_split(s1_ref[...])
    n1lo, n1hi = _tc_split(n1_ref[...])
    h = jnp.dot(s1lo, w1al_ref[...], preferred_element_type=f32)
    h = h + jnp.dot(s1hi, w1ah_ref[...], preferred_element_type=f32)
    h = h + jnp.dot(n1lo, w1bl_ref[...], preferred_element_type=f32)
    h = h + jnp.dot(n1hi, w1bh_ref[...], preferred_element_type=f32)
    h = jnp.maximum(h, 0.0)
    neigh2 = jnp.sum(h.reshape(_R, _S0, _D), axis=1)
    s0lo, s0hi = _tc_split(s0_ref[...])
    n0lo, n0hi = _tc_split(n0_ref[...])
    hs = jnp.dot(s0lo, w1al_ref[...], preferred_element_type=f32)
    hs = hs + jnp.dot(s0hi, w1ah_ref[...], preferred_element_type=f32)
    hs = hs + jnp.dot(n0lo, w1bl_ref[...], preferred_element_type=f32)
    hs = hs + jnp.dot(n0hi, w1bh_ref[...], preferred_element_type=f32)
    hs = jnp.maximum(hs, 0.0)
    logits = jnp.dot(hs, w2a_ref[...], preferred_element_type=f32)
    logits = logits + jnp.dot(neigh2, w2b_ref[...],
                              preferred_element_type=f32)
    m = jnp.max(logits, axis=-1, keepdims=True)
    e = jnp.exp(logits - m)
    out_ref[...] = e / jnp.sum(e, axis=-1, keepdims=True)


def _tc_dense(self1, neigh1, self0, neigh0,
              w1al, w1ah, w1bl, w1bh, w2a, w2b):
    grid = (_B // _R,)
    return pl.pallas_call(
        _tc_body,
        grid=grid,
        in_specs=[
            pl.BlockSpec((_R * _S0, _DP), lambda i: (i, 0)),
            pl.BlockSpec((_R * _S0, _DP), lambda i: (i, 0)),
            pl.BlockSpec((_R, _DP), lambda i: (i, 0)),
            pl.BlockSpec((_R, _DP), lambda i: (i, 0)),
            pl.BlockSpec((_DP, _D), lambda i: (0, 0)),
            pl.BlockSpec((_DP, _D), lambda i: (0, 0)),
            pl.BlockSpec((_DP, _D), lambda i: (0, 0)),
            pl.BlockSpec((_DP, _D), lambda i: (0, 0)),
            pl.BlockSpec((_D, _NCLASS), lambda i: (0, 0)),
            pl.BlockSpec((_D, _NCLASS), lambda i: (0, 0)),
        ],
        out_specs=pl.BlockSpec((_R, _NCLASS), lambda i: (i, 0)),
        out_shape=jax.ShapeDtypeStruct((_B, _NCLASS), jnp.float32),
    )(self1, neigh1, self0, neigh0, w1al, w1ah, w1bl, w1bh, w2a, w2b)


def kernel(feature, neighbor_array, train_node, W1, W2):
    fpk = _tc_pack(feature)
    n1f, n2idx = _sc_idx(neighbor_array, train_node)
    self1, neigh1, self0, neigh0 = _sc_feat(fpk, train_node, n1f, n2idx)

    w1b = W1[_D:] * (1.0 / _S1)
    w2b = W2[_D:] * (1.0 / _S0)
    bf = jnp.bfloat16
    return _tc_dense(self1, neigh1, self0, neigh0,
                     W1[:_DP].astype(bf), W1[_DP:_D].astype(bf),
                     w1b[:_DP].astype(bf), w1b[_DP:].astype(bf),
                     W2[:_D], w2b)


# speedup vs baseline: 1.0728x; 1.0146x over previous
import functools

import jax
import jax.numpy as jnp
from jax import lax
from jax.experimental import pallas as pl
from jax.experimental.pallas import tpu as pltpu
from jax.experimental.pallas import tpu_sc as plsc

_N, _D, _MAXDEG, _NCLASS, _B = 50000, 256, 32, 64, 1024
_S0, _S1 = 25, 10
_DP = _D // 2

_L = 16
_NC, _NS = 2, 16
_NW = _NC * _NS
_RPW = _B // _NW
_L1PW = _RPW * _S0
_GR = 8
_GL1 = _GR * _S0
_NGRP = _RPW // _GR
_CH2 = 80
_NB = 8
_FB = _NB * _S1


def _pipe2(n_blocks, fire, consume, bufA, semA, bufB, semB, wait):
    fire(0, bufA, semA)
    npairs = (n_blocks - 2) // 2

    def pair(i, c):
        fire(2 * i + 1, bufB, semB)
        wait(bufA, semA)
        consume(2 * i, bufA)
        fire(2 * i + 2, bufA, semA)
        wait(bufB, semB)
        consume(2 * i + 1, bufB)
        return c
    lax.fori_loop(0, npairs, pair, 0)
    k = 2 * npairs
    if n_blocks % 2 == 0:
        fire(n_blocks - 1, bufB, semB)
        wait(bufA, semA)
        consume(k, bufA)
        wait(bufB, semB)
        consume(n_blocks - 1, bufB)
    else:
        fire(n_blocks - 2, bufB, semB)
        wait(bufA, semA)
        consume(k, bufA)
        fire(n_blocks - 1, bufA, semA)
        wait(bufB, semB)
        consume(n_blocks - 2, bufB)
        wait(bufA, semA)
        consume(n_blocks - 1, bufA)


_KC = 40


def _row_fires(nbr_hbm, idx_v, idx_base, buf, sem, n):
    def fire(j, c):
        v = idx_v[pl.ds(idx_base + j, _L)][0]
        pltpu.async_copy(nbr_hbm.at[pl.ds(v, 1)], buf.at[pl.ds(j, 1)], sem)
        return c
    lax.fori_loop(0, n, fire, 0)


def _row_drain(nbr_hbm, buf, sem, n):
    def drain(j, c):
        pltpu.make_async_copy(nbr_hbm.at[pl.ds(0, 1)],
                              buf.at[pl.ds(0, 1)], sem).wait()
        return c
    lax.fori_loop(0, n, drain, 0)


def _sc_idx_body(nbr_hbm, tn_hbm, n1f_hbm, n2idx_hbm,
                 tn_v, n1rows_v, n1f_v, n2rA, n2rB, n2idx_v,
                 semA, semB):
    wid = lax.axis_index("s") * _NC + lax.axis_index("c")
    rbase = wid * _RPW

    pltpu.sync_copy(tn_hbm.at[pl.ds(rbase, _RPW)], tn_v.at[pl.ds(0, _RPW)])
    _row_fires(nbr_hbm, tn_v, 0, n1rows_v, semA, _RPW)
    _row_drain(nbr_hbm, n1rows_v, semA, _RPW)

    iota = lax.broadcasted_iota(jnp.int32, (_L,), 0)

    def pack25(i, c):
        k = i * _L + iota
        vals = plsc.load_gather(n1rows_v, [k // _S0, k % _S0])
        n1f_v[pl.ds(i * _L, _L)] = vals
        return c
    lax.fori_loop(0, _L1PW // _L, pack25, 0)
    pltpu.sync_copy(n1f_v.at[pl.ds(0, _L1PW)],
                    n1f_hbm.at[pl.ds(wid * _L1PW, _L1PW)])

    def n2_fire(m, buf, sem):
        _row_fires(nbr_hbm, n1f_v, m * _KC, buf, sem, _KC)

    def n2_wait(buf, sem):
        _row_drain(nbr_hbm, buf, sem, _KC)

    def n2_consume(m, buf):
        def pack10(i, c):
            k = i * _L + iota
            vals = plsc.load_gather(buf, [k // _S1, k % _S1])
            n2idx_v[pl.ds(m * _KC * _S1 + i * _L, _L)] = vals
            return c
        lax.fori_loop(0, _KC * _S1 // _L, pack10, 0)

    _pipe2(_L1PW // _KC, n2_fire, n2_consume, n2rA, semA, n2rB, semB,
           n2_wait)
    pltpu.sync_copy(n2idx_v,
                    n2idx_hbm.at[pl.ds(wid * _L1PW * _S1, _L1PW * _S1)])


_sc_idx = functools.partial(
    pl.kernel,
    out_type=(
        jax.ShapeDtypeStruct((_B * _S0,), jnp.int32),
        jax.ShapeDtypeStruct((_B * _S0 * _S1,), jnp.int32),
    ),
    mesh=plsc.VectorSubcoreMesh(core_axis_name="c", subcore_axis_name="s",
                                num_cores=_NC, num_subcores=_NS),
    compiler_params=pltpu.CompilerParams(needs_layout_passes=False),
    scratch_types=[
        pltpu.VMEM((_RPW + _L,), jnp.int32),
        pltpu.VMEM((_RPW, _MAXDEG), jnp.int32),
        pltpu.VMEM((_L1PW + _L,), jnp.int32),
        pltpu.VMEM((_KC, _MAXDEG), jnp.int32),
        pltpu.VMEM((_KC, _MAXDEG), jnp.int32),
        pltpu.VMEM((_L1PW * _S1,), jnp.int32),
        pltpu.SemaphoreType.DMA,
        pltpu.SemaphoreType.DMA,
    ],
)(_sc_idx_body)


def _acc_rows(src_ref, row0, nrows, dst_ref, dst_row):
    mask = jnp.full((_L,), -65536, dtype=jnp.int32)
    sh16 = jnp.full((_L,), 16, dtype=jnp.int32)
    for ch in range(_DP // _L):
        sl = pl.ds(ch * _L, _L)
        w = plsc.bitcast(src_ref[row0, sl], jnp.int32)
        acc_lo = plsc.bitcast(w << sh16, jnp.float32)
        acc_hi = plsc.bitcast(w & mask, jnp.float32)
        for c in range(1, nrows):
            w = plsc.bitcast(src_ref[row0 + c, sl], jnp.int32)
            acc_lo = acc_lo + plsc.bitcast(w << sh16, jnp.float32)
            acc_hi = acc_hi + plsc.bitcast(w & mask, jnp.float32)
        pk = plsc.pack(acc_lo, acc_hi, format=plsc.PackFormat.INTERLEAVED)
        dst_ref[dst_row, sl] = plsc.bitcast(pk, jnp.float32)


def _sc_feat_body(rofs, nroots,
                  feat_hbm, tn_hbm, n1f_hbm, n2idx_hbm,
                  self1_hbm, neigh1_hbm, self0_hbm, neigh0_hbm,
                  tn_v, n1f_v, n2idx_v, big_v, tmpA, tmpB, neigh0_v,
                  semA, semB, sem0):
    rpw = nroots // _NW
    l1pw = rpw * _S0
    ngrp = rpw // _GR
    wid = lax.axis_index("s") * _NC + lax.axis_index("c")
    grbase = rofs + wid * rpw
    rbase = wid * rpw
    _ns = jax.named_scope

    pltpu.sync_copy(tn_hbm.at[pl.ds(grbase, rpw)], tn_v)
    pltpu.async_copy(feat_hbm.at[tn_v], big_v.at[pl.ds(0, rpw)], sem0)
    pltpu.sync_copy(n1f_hbm.at[pl.ds(grbase * _S0, l1pw)], n1f_v)
    pltpu.sync_copy(n2idx_hbm.at[pl.ds(grbase * _S0 * _S1, l1pw * _S1)],
                    n2idx_v)

    pltpu.make_async_copy(feat_hbm.at[pl.ds(0, rpw)],
                          big_v.at[pl.ds(0, rpw)], sem0).wait()
    pltpu.sync_copy(big_v.at[pl.ds(0, rpw)], self0_hbm.at[pl.ds(rbase, rpw)])

    def do_group(g, c):
        lbase = g * _GL1
        growbase = (rbase + g * _GR) * _S0

        pltpu.async_copy(feat_hbm.at[n1f_v.at[pl.ds(lbase, 120)]],
                         big_v.at[pl.ds(0, 120)], semA)
        cp2 = pltpu.async_copy(feat_hbm.at[n1f_v.at[pl.ds(lbase + 120, 80)]],
                               big_v.at[pl.ds(120, 80)], semB)
        with _ns("self1wait"):
            pltpu.make_async_copy(feat_hbm.at[pl.ds(0, 120)],
                                  big_v.at[pl.ds(0, 120)], semA).wait()
            cp2.wait()
            pltpu.sync_copy(big_v, self1_hbm.at[pl.ds(growbase, _GL1)])

        def n0root(r, cc):
            _acc_rows(big_v, r * _S0, _S1, neigh0_v, g * _GR + r)
            return cc
        with _ns("n0acc"):
            lax.fori_loop(0, _GR, n0root, 0)

        def n1_fire(t, buf, sem):
            pltpu.async_copy(
                feat_hbm.at[n2idx_v.at[pl.ds((lbase + t * _NB) * _S1, _FB)]],
                buf, sem)

        def n1_wait(buf, sem):
            pltpu.make_async_copy(feat_hbm.at[pl.ds(0, _FB)], buf, sem).wait()

        def n1_consume(t, buf):
            def node(nn, cc):
                _acc_rows(buf, nn * _S1, _S1, big_v, t * _NB + nn)
                return cc
            lax.fori_loop(0, _NB, node, 0)

        with _ns("n1pipe"):
            _pipe2(_GL1 // _NB, n1_fire, n1_consume, tmpA, semA, tmpB, semB,
                   n1_wait)
        with _ns("n1flush"):
            pltpu.sync_copy(big_v, neigh1_hbm.at[pl.ds(growbase, _GL1)])
        return c
    lax.fori_loop(0, ngrp, do_group, 0)

    pltpu.sync_copy(neigh0_v, neigh0_hbm.at[pl.ds(rbase, rpw)])


def _mk_sc_feat(rofs, nroots):
    rpw = nroots // _NW
    return functools.partial(
        pl.kernel,
        out_type=(
            jax.ShapeDtypeStruct((nroots * _S0, _DP), jnp.float32),
            jax.ShapeDtypeStruct((nroots * _S0, _DP), jnp.float32),
            jax.ShapeDtypeStruct((nroots, _DP), jnp.float32),
            jax.ShapeDtypeStruct((nroots, _DP), jnp.float32),
        ),
        mesh=plsc.VectorSubcoreMesh(core_axis_name="c", subcore_axis_name="s",
                                    num_cores=_NC, num_subcores=_NS),
        compiler_params=pltpu.CompilerParams(needs_layout_passes=False),
        scratch_types=[
            pltpu.VMEM((rpw,), jnp.int32),
            pltpu.VMEM((rpw * _S0,), jnp.int32),
            pltpu.VMEM((rpw * _S0 * _S1,), jnp.int32),
            pltpu.VMEM((_GL1, _DP), jnp.float32),
            pltpu.VMEM((_FB, _DP), jnp.float32),
            pltpu.VMEM((_FB, _DP), jnp.float32),
            pltpu.VMEM((rpw, _DP), jnp.float32),
            pltpu.SemaphoreType.DMA,
            pltpu.SemaphoreType.DMA,
            pltpu.SemaphoreType.DMA,
        ],
    )(functools.partial(_sc_feat_body, rofs, nroots))


_HB = _B // 2
_sc_feat_a = _mk_sc_feat(0, _HB)
_sc_feat_b = _mk_sc_feat(_HB, _HB)


_PKROWS = 2000


def _tc_pack_body(f_ref, out_ref):
    bits = lax.bitcast_convert_type(f_ref[...], jnp.int32)
    rnd = bits + jnp.int32(0x7FFF) + ((bits >> 16) & jnp.int32(1))
    lo = (rnd[:, :_DP] >> 16) & jnp.int32(0xFFFF)
    hi = rnd[:, _DP:] & jnp.int32(-65536)
    out_ref[...] = lax.bitcast_convert_type(lo | hi, jnp.float32)


def _tc_pack(feature):
    return pl.pallas_call(
        _tc_pack_body,
        grid=(_N // _PKROWS,),
        in_specs=[pl.BlockSpec((_PKROWS, _D), lambda i: (i, 0))],
        out_specs=pl.BlockSpec((_PKROWS, _DP), lambda i: (i, 0)),
        out_shape=jax.ShapeDtypeStruct((_N, _DP), jnp.float32),
    )(feature)


_R = 128


def _tc_split(x):
    b = lax.bitcast_convert_type(x, jnp.int32)
    lo = lax.bitcast_convert_type(b << 16, jnp.float32)
    hi = lax.bitcast_convert_type(b & jnp.int32(-65536), jnp.float32)
    return lo.astype(jnp.bfloat16), hi.astype(jnp.bfloat16)


def _tc_body(s1_ref, n1_ref, s0_ref, n0_ref,
             w1al_ref, w1ah_ref, w1bl_ref, w1bh_ref, w2a_ref, w2b_ref,
             out_ref):
    f32 = jnp.float32
    s1lo, s1hi = _tc_split(s1_ref[...])
    n1lo, n1hi = _tc_split(n1_ref[...])
    h = jnp.dot(s1lo, w1al_ref[...], preferred_element_type=f32)
    h = h + jnp.dot(s1hi, w1ah_ref[...], preferred_element_type=f32)
    h = h + jnp.dot(n1lo, w1bl_ref[...], preferred_element_type=f32)
    h = h + jnp.dot(n1hi, w1bh_ref[...], preferred_element_type=f32)
    h = jnp.maximum(h, 0.0)
    neigh2 = jnp.sum(h.reshape(_R, _S0, _D), axis=1)
    s0lo, s0hi = _tc_split(s0_ref[...])
    n0lo, n0hi = _tc_split(n0_ref[...])
    hs = jnp.dot(s0lo, w1al_ref[...], preferred_element_type=f32)
    hs = hs + jnp.dot(s0hi, w1ah_ref[...], preferred_element_type=f32)
    hs = hs + jnp.dot(n0lo, w1bl_ref[...], preferred_element_type=f32)
    hs = hs + jnp.dot(n0hi, w1bh_ref[...], preferred_element_type=f32)
    hs = jnp.maximum(hs, 0.0)
    logits = jnp.dot(hs, w2a_ref[...], preferred_element_type=f32)
    logits = logits + jnp.dot(neigh2, w2b_ref[...],
                              preferred_element_type=f32)
    m = jnp.max(logits, axis=-1, keepdims=True)
    e = jnp.exp(logits - m)
    out_ref[...] = e / jnp.sum(e, axis=-1, keepdims=True)


def _tc_dense(self1, neigh1, self0, neigh0,
              w1al, w1ah, w1bl, w1bh, w2a, w2b):
    nroots = self0.shape[0]
    return pl.pallas_call(
        _tc_body,
        grid=(nroots // _R,),
        in_specs=[
            pl.BlockSpec((_R * _S0, _DP), lambda i: (i, 0)),
            pl.BlockSpec((_R * _S0, _DP), lambda i: (i, 0)),
            pl.BlockSpec((_R, _DP), lambda i: (i, 0)),
            pl.BlockSpec((_R, _DP), lambda i: (i, 0)),
            pl.BlockSpec((_DP, _D), lambda i: (0, 0)),
            pl.BlockSpec((_DP, _D), lambda i: (0, 0)),
            pl.BlockSpec((_DP, _D), lambda i: (0, 0)),
            pl.BlockSpec((_DP, _D), lambda i: (0, 0)),
            pl.BlockSpec((_D, _NCLASS), lambda i: (0, 0)),
            pl.BlockSpec((_D, _NCLASS), lambda i: (0, 0)),
        ],
        out_specs=pl.BlockSpec((_R, _NCLASS), lambda i: (i, 0)),
        out_shape=jax.ShapeDtypeStruct((nroots, _NCLASS), jnp.float32),
    )(self1, neigh1, self0, neigh0, w1al, w1ah, w1bl, w1bh, w2a, w2b)


def kernel(feature, neighbor_array, train_node, W1, W2):
    fpk = _tc_pack(feature)
    n1f, n2idx = _sc_idx(neighbor_array, train_node)

    w1b = W1[_D:] * (1.0 / _S1)
    w2b = W2[_D:] * (1.0 / _S0)
    bf = jnp.bfloat16
    ws = (W1[:_DP].astype(bf), W1[_DP:_D].astype(bf),
          w1b[:_DP].astype(bf), w1b[_DP:].astype(bf),
          W2[:_D], w2b)
    ha = _sc_feat_a(fpk, train_node, n1f, n2idx)
    hb = _sc_feat_b(fpk, train_node, n1f, n2idx)
    out_a = _tc_dense(*ha, *ws)
    out_b = _tc_dense(*hb, *ws)
    return jnp.concatenate([out_a, out_b], axis=0)
